# Initial kernel scaffold; baseline (speedup 1.0000x reference)
import jax
import jax.numpy as jnp
from jax.experimental import pallas as pl


def kernel(ranks_logits, dia_lens):
    raise NotImplementedError("write your pallas kernel here")


# speedup vs baseline: 4.7716x; 4.7716x over previous
import functools

import jax
import jax.numpy as jnp
from jax import lax
from jax.experimental import pallas as pl
from jax.experimental.pallas import tpu as pltpu
from jax.experimental.pallas import tpu_sc as plsc

_B = 16
_T = 4096
_CH = 16
_NCHUNKS = _T // _CH
_NW = 32
_CPW = _NCHUNKS // _NW


def _sc_counts(x_flat, dia_lens):
    mesh = plsc.VectorSubcoreMesh(core_axis_name="c", subcore_axis_name="s")

    @functools.partial(
        pl.kernel,
        mesh=mesh,
        out_type=jax.ShapeDtypeStruct((_NW, _B * _CH), jnp.int32),
        scratch_types=[
            pltpu.VMEM((_B * _T,), jnp.float32),
            pltpu.VMEM((_B,), jnp.int32),
            pltpu.VMEM((_B * _CH,), jnp.int32),
        ],
    )
    def k(x_hbm, lens_hbm, out_hbm, x_v, lens_v, acc_v):
        wid = lax.axis_index("s") * 2 + lax.axis_index("c")
        pltpu.sync_copy(x_hbm, x_v)
        pltpu.sync_copy(lens_hbm, lens_v)
        iota = lax.iota(jnp.int32, _CH)
        neg_inf = jnp.float32(-jnp.inf)
        zeros = jnp.zeros((_CH,), jnp.int32)
        lreg = lens_v[...]

        for row in range(_B):
            L = lreg[row]
            Lv = jnp.full((_CH,), L, jnp.int32)
            ceil_chunks = (L + _CH - 1) // _CH

            def unit_body(jj, tot, row=row, L=L, Lv=Lv,
                          ceil_chunks=ceil_chunks):
                c_lin = wid + _NW * jj
                if row % 2 == 1:
                    c = _NCHUNKS - 1 - c_lin
                else:
                    c = c_lin
                base = c * _CH
                jmax = jnp.minimum(c, ceil_chunks)

                va = x_v[pl.ds(row * _T + base, _CH)]
                va_m = jnp.where(base + iota < Lv, va, neg_inf)
                bs = [jnp.full((_CH,), va_m[i], jnp.float32)
                      for i in range(_CH)]
                accs = [zeros for _ in range(_CH)]
                for i in range(1, _CH):
                    m = (iota < i) & (va_m < bs[i])
                    accs[i] = accs[i] + jnp.where(m, 1, 0)

                def jbody(j, acc_t):
                    vb = x_v[pl.ds(row * _T + j * _CH, _CH)]
                    return tuple(a + jnp.where(vb < b, 1, 0)
                                 for a, b in zip(acc_t, bs))

                accs2 = list(lax.fori_loop(0, jmax, jbody, tuple(accs)))
                while len(accs2) > 1:
                    nxt = [accs2[2 * i] + accs2[2 * i + 1]
                           for i in range(len(accs2) // 2)]
                    if len(accs2) % 2:
                        nxt.append(accs2[-1])
                    accs2 = nxt
                return tot + accs2[0]

            tot = lax.fori_loop(0, _CPW, unit_body, zeros)
            acc_v[pl.ds(row * _CH, _CH)] = tot

        pltpu.sync_copy(acc_v, out_hbm.at[wid])

    return k(x_flat, dia_lens)


def _tc_finalize(x, lens_col, counts):
    def body(x_ref, lc_ref, cnt_ref, loss_ref, si_ref):
        xv = x_ref[...]
        lens = lc_ref[...]
        pos = lax.broadcasted_iota(jnp.int32, (_B, _T), 1)
        mask = pos >= lens
        lf = lens.astype(jnp.float32)
        lin = pos.astype(jnp.float32) / (lf - 1.0)
        padded = jnp.where(mask, jnp.float32(1.0), lin)
        q = 2.0 * padded
        q2 = q * q
        q5 = q2 * q2 * q
        rt = 1.0 / (1.0 + q5)
        ml = jnp.where(mask, -jnp.inf, xv)
        kl = rt * (jnp.log(rt) - ml)
        loss_ref[0, 0] = jnp.sum(kl) / jnp.float32(_B)

        inv = jnp.sum(cnt_ref[...].astype(jnp.float32), axis=1, keepdims=True)
        max_inv = lf * (lf - 1.0) * 0.5
        total = jnp.sum(inv / max_inv)
        si_ref[0, 0] = 1.0 - total / jnp.float32(_B)

    loss, si = pl.pallas_call(
        body,
        out_shape=[
            jax.ShapeDtypeStruct((1, 1), jnp.float32),
            jax.ShapeDtypeStruct((1, 1), jnp.float32),
        ],
        out_specs=[
            pl.BlockSpec(memory_space=pltpu.SMEM),
            pl.BlockSpec(memory_space=pltpu.SMEM),
        ],
    )(x, lens_col, counts)
    return loss[0, 0], si[0, 0]


def kernel(ranks_logits, dia_lens):
    counts = _sc_counts(ranks_logits.reshape(-1), dia_lens)
    counts = counts.reshape(_NW, _B, _CH).transpose(1, 0, 2).reshape(_B, -1)
    loss, si = _tc_finalize(ranks_logits, dia_lens.reshape(_B, 1), counts)
    return (loss, si)


# speedup vs baseline: 4.7741x; 1.0005x over previous
import functools

import jax
import jax.numpy as jnp
from jax import lax
from jax.experimental import pallas as pl
from jax.experimental.pallas import tpu as pltpu
from jax.experimental.pallas import tpu_sc as plsc

_B = 16
_T = 4096
_CH = 16
_NCHUNKS = _T // _CH
_NW = 32
_CPW = _NCHUNKS // _NW


def _sc_counts(x_flat, dia_lens):
    mesh = plsc.VectorSubcoreMesh(core_axis_name="c", subcore_axis_name="s")

    @functools.partial(
        pl.kernel,
        mesh=mesh,
        out_type=jax.ShapeDtypeStruct((_NW, _B * _CH), jnp.int32),
        scratch_types=[
            pltpu.VMEM((_B * _T,), jnp.float32),
            pltpu.VMEM((_B,), jnp.int32),
            pltpu.VMEM((_B * _CH,), jnp.int32),
        ],
    )
    def k(x_hbm, lens_hbm, out_hbm, x_v, lens_v, acc_v):
        wid = lax.axis_index("s") * 2 + lax.axis_index("c")
        pltpu.sync_copy(x_hbm, x_v)
        pltpu.sync_copy(lens_hbm, lens_v)
        iota = lax.iota(jnp.int32, _CH)
        neg_inf = jnp.float32(-jnp.inf)
        zeros = jnp.zeros((_CH,), jnp.int32)
        lreg = lens_v[...]

        for row in range(_B):
            L = lreg[row]
            Lv = jnp.full((_CH,), L, jnp.int32)
            ceil_chunks = (L + _CH - 1) // _CH

            def unit_body(jj, tot, row=row, L=L, Lv=Lv,
                          ceil_chunks=ceil_chunks):
                c_lin = wid + _NW * jj
                if row % 2 == 1:
                    c = _NCHUNKS - 1 - c_lin
                else:
                    c = c_lin
                base = c * _CH
                jmax = jnp.minimum(c, ceil_chunks)

                va = x_v[pl.ds(row * _T + base, _CH)]
                va_m = jnp.where(base + iota < Lv, va, neg_inf)
                bs = [jnp.full((_CH,), va_m[i], jnp.float32)
                      for i in range(_CH)]
                accs = [zeros for _ in range(_CH)]
                for i in range(1, _CH):
                    m = (iota < i) & (va_m < bs[i])
                    accs[i] = accs[i] + jnp.where(m, 1, 0)

                def jloop(j, acc_t):
                    vb = x_v[pl.ds(row * _T + j * _CH, _CH)]
                    return tuple(a + jnp.where(vb < b, 1, 0)
                                 for a, b in zip(acc_t, bs))

                accs2 = list(
                    plsc.parallel_loop(0, jmax, 1, unroll=4,
                                       carry=tuple(accs))(jloop))
                while len(accs2) > 1:
                    nxt = [accs2[2 * i] + accs2[2 * i + 1]
                           for i in range(len(accs2) // 2)]
                    if len(accs2) % 2:
                        nxt.append(accs2[-1])
                    accs2 = nxt
                return tot + accs2[0]

            tot = lax.fori_loop(0, _CPW, unit_body, zeros)
            acc_v[pl.ds(row * _CH, _CH)] = tot

        pltpu.sync_copy(acc_v, out_hbm.at[wid])

    return k(x_flat, dia_lens)


def _tc_finalize(x, lens_col, counts):
    def body(x_ref, lc_ref, cnt_ref, loss_ref, si_ref):
        xv = x_ref[...]
        lens = lc_ref[...]
        pos = lax.broadcasted_iota(jnp.int32, (_B, _T), 1)
        mask = pos >= lens
        lf = lens.astype(jnp.float32)
        lin = pos.astype(jnp.float32) / (lf - 1.0)
        padded = jnp.where(mask, jnp.float32(1.0), lin)
        q = 2.0 * padded
        q2 = q * q
        q5 = q2 * q2 * q
        rt = 1.0 / (1.0 + q5)
        ml = jnp.where(mask, -jnp.inf, xv)
        kl = rt * (jnp.log(rt) - ml)
        loss_ref[0, 0] = jnp.sum(kl) / jnp.float32(_B)

        inv = jnp.sum(cnt_ref[...].astype(jnp.float32), axis=1, keepdims=True)
        max_inv = lf * (lf - 1.0) * 0.5
        total = jnp.sum(inv / max_inv)
        si_ref[0, 0] = 1.0 - total / jnp.float32(_B)

    loss, si = pl.pallas_call(
        body,
        out_shape=[
            jax.ShapeDtypeStruct((1, 1), jnp.float32),
            jax.ShapeDtypeStruct((1, 1), jnp.float32),
        ],
        out_specs=[
            pl.BlockSpec(memory_space=pltpu.SMEM),
            pl.BlockSpec(memory_space=pltpu.SMEM),
        ],
    )(x, lens_col, counts)
    return loss[0, 0], si[0, 0]


def kernel(ranks_logits, dia_lens):
    counts = _sc_counts(ranks_logits.reshape(-1), dia_lens)
    counts = counts.reshape(_NW, _B, _CH).transpose(1, 0, 2).reshape(_B, -1)
    loss, si = _tc_finalize(ranks_logits, dia_lens.reshape(_B, 1), counts)
    return (loss, si)
